# initial kernel scaffold (unmeasured)
import jax
import jax.numpy as jnp
from jax import lax
from jax.experimental import pallas as pl
from jax.experimental.pallas import tpu as pltpu

N_DEV = 8
M_PER = 1024
D = 1024


def kernel(partial, gamma):
    gamma2 = gamma.reshape(1, D)

    def body(p_ref, g_ref, out_ref, send_buf, recv_buf, send_sems, recv_sems):
        my = lax.axis_index("i")
        left = lax.rem(my + N_DEV - 1, N_DEV)
        right = lax.rem(my + 1, N_DEV)

        barrier_sem = pltpu.get_barrier_semaphore()
        for nbr in (left, right):
            pl.semaphore_signal(
                barrier_sem, inc=1,
                device_id=(nbr,), device_id_type=pl.DeviceIdType.MESH,
            )
        pl.semaphore_wait(barrier_sem, 2)

        def chunk(c):
            return p_ref[0, pl.ds(c * M_PER, M_PER), :]

        send_buf[0] = chunk(left).astype(jnp.bfloat16)

        for h in range(N_DEV - 1):
            rdma = pltpu.make_async_remote_copy(
                src_ref=send_buf.at[h],
                dst_ref=recv_buf.at[h],
                send_sem=send_sems.at[h],
                recv_sem=recv_sems.at[h],
                device_id=(right,),
                device_id_type=pl.DeviceIdType.MESH,
            )
            rdma.start()
            rdma.wait()
            c = lax.rem(my + (2 * N_DEV - h - 2), N_DEV)
            if h < N_DEV - 2:
                send_buf[h + 1] = recv_buf[h] + chunk(c).astype(jnp.bfloat16)
            else:
                y = recv_buf[h][...].astype(jnp.float32) + chunk(c)
                rms = jnp.sqrt(jnp.mean(y * y, axis=-1, keepdims=True) + 1e-6)
                out_ref[...] = y / rms * g_ref[...]

    return pl.pallas_call(
        body,
        out_shape=jax.ShapeDtypeStruct((M_PER, D), jnp.float32),
        in_specs=[
            pl.BlockSpec(memory_space=pltpu.VMEM),
            pl.BlockSpec(memory_space=pltpu.VMEM),
        ],
        out_specs=pl.BlockSpec(memory_space=pltpu.VMEM),
        scratch_shapes=[
            pltpu.VMEM((N_DEV - 1, M_PER, D), jnp.bfloat16),
            pltpu.VMEM((N_DEV - 1, M_PER, D), jnp.bfloat16),
            pltpu.SemaphoreType.DMA((N_DEV - 1,)),
            pltpu.SemaphoreType.DMA((N_DEV - 1,)),
        ],
        compiler_params=pltpu.CompilerParams(collective_id=0),
    )(partial, gamma2)


# baseline (device time: 185905 ns/iter reference)
import jax
import jax.numpy as jnp
from jax import lax
from jax.experimental import pallas as pl
from jax.experimental.pallas import tpu as pltpu

N_DEV = 8
M_PER = 1024
D = 1024


def kernel(partial, gamma):
    gamma2 = gamma.reshape(1, D)

    def body(p_ref, g_ref, out_ref, send_buf, recv_buf, stage,
             send_sems, recv_sems, copy_sems):
        my = lax.axis_index("i")
        left = lax.rem(my + N_DEV - 1, N_DEV)
        right = lax.rem(my + 1, N_DEV)

        barrier_sem = pltpu.get_barrier_semaphore()
        for nbr in (left, right):
            pl.semaphore_signal(
                barrier_sem, inc=1,
                device_id=(nbr,), device_id_type=pl.DeviceIdType.MESH,
            )
        pl.semaphore_wait(barrier_sem, 2)

        def fetch_chunk(c, slot):
            return pltpu.make_async_copy(
                p_ref.at[0, pl.ds(c * M_PER, M_PER), :],
                stage.at[slot],
                copy_sems.at[slot],
            )

        cp = fetch_chunk(left, 0)
        cp.start()
        cp.wait()
        send_buf[0] = stage[0].astype(jnp.bfloat16)
        fetch_chunk(lax.rem(my + 2 * N_DEV - 2, N_DEV), 1).start()

        for h in range(N_DEV - 1):
            acc_slot = (h + 1) % 2
            rdma = pltpu.make_async_remote_copy(
                src_ref=send_buf.at[h],
                dst_ref=recv_buf.at[h],
                send_sem=send_sems.at[h],
                recv_sem=recv_sems.at[h],
                device_id=(right,),
                device_id_type=pl.DeviceIdType.MESH,
            )
            rdma.start()
            pltpu.make_async_copy(
                stage.at[acc_slot], stage.at[acc_slot], copy_sems.at[acc_slot]
            ).wait()
            if h < N_DEV - 2:
                fetch_chunk(lax.rem(my + 2 * N_DEV - h - 3, N_DEV), h % 2).start()
            rdma.wait()
            if h < N_DEV - 2:
                send_buf[h + 1] = (
                    recv_buf[h] + stage[acc_slot].astype(jnp.bfloat16)
                )
            else:
                y = recv_buf[h][...].astype(jnp.float32) + stage[acc_slot]
                rms = jnp.sqrt(jnp.mean(y * y, axis=-1, keepdims=True) + 1e-6)
                out_ref[...] = y / rms * g_ref[...]

    return pl.pallas_call(
        body,
        out_shape=jax.ShapeDtypeStruct((M_PER, D), jnp.float32),
        in_specs=[
            pl.BlockSpec(memory_space=pl.ANY),
            pl.BlockSpec(memory_space=pltpu.VMEM),
        ],
        out_specs=pl.BlockSpec(memory_space=pltpu.VMEM),
        scratch_shapes=[
            pltpu.VMEM((N_DEV - 1, M_PER, D), jnp.bfloat16),
            pltpu.VMEM((N_DEV - 1, M_PER, D), jnp.bfloat16),
            pltpu.VMEM((2, M_PER, D), jnp.float32),
            pltpu.SemaphoreType.DMA((N_DEV - 1,)),
            pltpu.SemaphoreType.DMA((N_DEV - 1,)),
            pltpu.SemaphoreType.DMA((2,)),
        ],
        compiler_params=pltpu.CompilerParams(
            collective_id=0,
            vmem_limit_bytes=100 * 1024 * 1024,
        ),
    )(partial, gamma2)


# device time: 109242 ns/iter; 1.7018x vs baseline; 1.7018x over previous
import jax
import jax.numpy as jnp
from jax import lax
from jax.experimental import pallas as pl
from jax.experimental.pallas import tpu as pltpu

N_DEV = 8
M_PER = 1024
H = M_PER // 2
D = 1024


def kernel(partial, gamma):
    gamma2 = gamma.reshape(1, D)

    def body(p_ref, g_ref, out_ref,
             send_r, recv_r, send_l, recv_l, stage_r, stage_l,
             ssem_r, rsem_r, ssem_l, rsem_l, csem_r, csem_l):
        my = lax.axis_index("i")
        left = lax.rem(my + N_DEV - 1, N_DEV)
        right = lax.rem(my + 1, N_DEV)

        barrier_sem = pltpu.get_barrier_semaphore()
        for nbr in (left, right):
            pl.semaphore_signal(
                barrier_sem, inc=1,
                device_id=(nbr,), device_id_type=pl.DeviceIdType.MESH,
            )
        pl.semaphore_wait(barrier_sem, 2)

        def fetch_top(c, slot):
            return pltpu.make_async_copy(
                p_ref.at[0, pl.ds(c * M_PER, H), :],
                stage_r.at[slot], csem_r.at[slot],
            )

        def fetch_bot(c, slot):
            return pltpu.make_async_copy(
                p_ref.at[0, pl.ds(c * M_PER + H, H), :],
                stage_l.at[slot], csem_l.at[slot],
            )

        cp_r = fetch_top(left, 0)
        cp_l = fetch_bot(right, 0)
        cp_r.start()
        cp_l.start()
        cp_r.wait()
        send_r[0] = stage_r[0].astype(jnp.bfloat16)
        cp_l.wait()
        send_l[0] = stage_l[0].astype(jnp.bfloat16)
        fetch_top(lax.rem(my + 2 * N_DEV - 2, N_DEV), 1).start()
        fetch_bot(lax.rem(my + 2, N_DEV), 1).start()

        for h in range(N_DEV - 1):
            acc_slot = (h + 1) % 2
            rdma_r = pltpu.make_async_remote_copy(
                src_ref=send_r.at[h], dst_ref=recv_r.at[h],
                send_sem=ssem_r.at[h], recv_sem=rsem_r.at[h],
                device_id=(right,), device_id_type=pl.DeviceIdType.MESH,
            )
            rdma_l = pltpu.make_async_remote_copy(
                src_ref=send_l.at[h], dst_ref=recv_l.at[h],
                send_sem=ssem_l.at[h], recv_sem=rsem_l.at[h],
                device_id=(left,), device_id_type=pl.DeviceIdType.MESH,
            )
            rdma_r.start()
            rdma_l.start()
            pltpu.make_async_copy(
                stage_r.at[acc_slot], stage_r.at[acc_slot], csem_r.at[acc_slot]
            ).wait()
            pltpu.make_async_copy(
                stage_l.at[acc_slot], stage_l.at[acc_slot], csem_l.at[acc_slot]
            ).wait()
            if h < N_DEV - 2:
                fetch_top(lax.rem(my + 2 * N_DEV - h - 3, N_DEV), h % 2).start()
                fetch_bot(lax.rem(my + h + 3, N_DEV), h % 2).start()
            rdma_r.wait()
            rdma_l.wait()
            if h < N_DEV - 2:
                send_r[h + 1] = recv_r[h] + stage_r[acc_slot].astype(jnp.bfloat16)
                send_l[h + 1] = recv_l[h] + stage_l[acc_slot].astype(jnp.bfloat16)
            else:
                y_t = recv_r[h][...].astype(jnp.float32) + stage_r[acc_slot]
                y_b = recv_l[h][...].astype(jnp.float32) + stage_l[acc_slot]
                rms_t = jnp.sqrt(jnp.mean(y_t * y_t, axis=-1, keepdims=True) + 1e-6)
                rms_b = jnp.sqrt(jnp.mean(y_b * y_b, axis=-1, keepdims=True) + 1e-6)
                out_ref[pl.ds(0, H), :] = y_t / rms_t * g_ref[...]
                out_ref[pl.ds(H, H), :] = y_b / rms_b * g_ref[...]

    return pl.pallas_call(
        body,
        out_shape=jax.ShapeDtypeStruct((M_PER, D), jnp.float32),
        in_specs=[
            pl.BlockSpec(memory_space=pl.ANY),
            pl.BlockSpec(memory_space=pltpu.VMEM),
        ],
        out_specs=pl.BlockSpec(memory_space=pltpu.VMEM),
        scratch_shapes=[
            pltpu.VMEM((N_DEV - 1, H, D), jnp.bfloat16),
            pltpu.VMEM((N_DEV - 1, H, D), jnp.bfloat16),
            pltpu.VMEM((N_DEV - 1, H, D), jnp.bfloat16),
            pltpu.VMEM((N_DEV - 1, H, D), jnp.bfloat16),
            pltpu.VMEM((2, H, D), jnp.float32),
            pltpu.VMEM((2, H, D), jnp.float32),
            pltpu.SemaphoreType.DMA((N_DEV - 1,)),
            pltpu.SemaphoreType.DMA((N_DEV - 1,)),
            pltpu.SemaphoreType.DMA((N_DEV - 1,)),
            pltpu.SemaphoreType.DMA((N_DEV - 1,)),
            pltpu.SemaphoreType.DMA((2,)),
            pltpu.SemaphoreType.DMA((2,)),
        ],
        compiler_params=pltpu.CompilerParams(
            collective_id=0,
            vmem_limit_bytes=100 * 1024 * 1024,
        ),
    )(partial, gamma2)


# device time: 94879 ns/iter; 1.9594x vs baseline; 1.1514x over previous
import jax
import jax.numpy as jnp
from jax import lax
from jax.experimental import pallas as pl
from jax.experimental.pallas import tpu as pltpu

N_DEV = 8
M_PER = 1024
H = M_PER // 2
K = 2
SB = H // K
D = 1024


def kernel(partial, gamma):
    gamma2 = gamma.reshape(1, D)

    def body(p_ref, g_ref, out_ref,
             send_r, recv_r, send_l, recv_l, stage_r, stage_l,
             ssem_r, rsem_r, ssem_l, rsem_l, csem_r, csem_l):
        my = lax.axis_index("i")
        left = lax.rem(my + N_DEV - 1, N_DEV)
        right = lax.rem(my + 1, N_DEV)

        barrier_sem = pltpu.get_barrier_semaphore()
        for nbr in (left, right):
            pl.semaphore_signal(
                barrier_sem, inc=1,
                device_id=(nbr,), device_id_type=pl.DeviceIdType.MESH,
            )
        pl.semaphore_wait(barrier_sem, 2)

        def fetch_top(c, slot):
            return pltpu.make_async_copy(
                p_ref.at[0, pl.ds(c * M_PER, H), :],
                stage_r.at[slot], csem_r.at[slot],
            )

        def fetch_bot(c, slot):
            return pltpu.make_async_copy(
                p_ref.at[0, pl.ds(c * M_PER + H, H), :],
                stage_l.at[slot], csem_l.at[slot],
            )

        def rdma(h, k, sbuf, rbuf, ssem, rsem, dev):
            return pltpu.make_async_remote_copy(
                src_ref=sbuf.at[h, pl.ds(k * SB, SB), :],
                dst_ref=rbuf.at[h, pl.ds(k * SB, SB), :],
                send_sem=ssem.at[h, k], recv_sem=rsem.at[h, k],
                device_id=(dev,), device_id_type=pl.DeviceIdType.MESH,
            )

        cp_r = fetch_top(left, 0)
        cp_l = fetch_bot(right, 0)
        cp_r.start()
        cp_l.start()
        cp_r.wait()
        send_r[0] = stage_r[0].astype(jnp.bfloat16)
        for k in range(K):
            rdma(0, k, send_r, recv_r, ssem_r, rsem_r, right).start()
        cp_l.wait()
        send_l[0] = stage_l[0].astype(jnp.bfloat16)
        for k in range(K):
            rdma(0, k, send_l, recv_l, ssem_l, rsem_l, left).start()
        fetch_top(lax.rem(my + 2 * N_DEV - 2, N_DEV), 1).start()
        fetch_bot(lax.rem(my + 2, N_DEV), 1).start()

        for h in range(N_DEV - 1):
            acc_slot = (h + 1) % 2
            pltpu.make_async_copy(
                stage_r.at[acc_slot], stage_r.at[acc_slot], csem_r.at[acc_slot]
            ).wait()
            pltpu.make_async_copy(
                stage_l.at[acc_slot], stage_l.at[acc_slot], csem_l.at[acc_slot]
            ).wait()
            if h < N_DEV - 2:
                fetch_top(lax.rem(my + 2 * N_DEV - h - 3, N_DEV), h % 2).start()
                fetch_bot(lax.rem(my + h + 3, N_DEV), h % 2).start()
            for k in range(K):
                ds_k = pl.ds(k * SB, SB)
                rdma(h, k, send_r, recv_r, ssem_r, rsem_r, right).wait()
                if h < N_DEV - 2:
                    send_r[h + 1, ds_k, :] = (
                        recv_r[h, ds_k, :]
                        + stage_r[acc_slot, ds_k, :].astype(jnp.bfloat16)
                    )
                    rdma(h + 1, k, send_r, recv_r, ssem_r, rsem_r, right).start()
                else:
                    y = (recv_r[h, ds_k, :].astype(jnp.float32)
                         + stage_r[acc_slot, ds_k, :])
                    rms = jnp.sqrt(
                        jnp.mean(y * y, axis=-1, keepdims=True) + 1e-6)
                    out_ref[pl.ds(k * SB, SB), :] = y / rms * g_ref[...]
                rdma(h, k, send_l, recv_l, ssem_l, rsem_l, left).wait()
                if h < N_DEV - 2:
                    send_l[h + 1, ds_k, :] = (
                        recv_l[h, ds_k, :]
                        + stage_l[acc_slot, ds_k, :].astype(jnp.bfloat16)
                    )
                    rdma(h + 1, k, send_l, recv_l, ssem_l, rsem_l, left).start()
                else:
                    y = (recv_l[h, ds_k, :].astype(jnp.float32)
                         + stage_l[acc_slot, ds_k, :])
                    rms = jnp.sqrt(
                        jnp.mean(y * y, axis=-1, keepdims=True) + 1e-6)
                    out_ref[pl.ds(H + k * SB, SB), :] = y / rms * g_ref[...]

    return pl.pallas_call(
        body,
        out_shape=jax.ShapeDtypeStruct((M_PER, D), jnp.float32),
        in_specs=[
            pl.BlockSpec(memory_space=pl.ANY),
            pl.BlockSpec(memory_space=pltpu.VMEM),
        ],
        out_specs=pl.BlockSpec(memory_space=pltpu.VMEM),
        scratch_shapes=[
            pltpu.VMEM((N_DEV - 1, H, D), jnp.bfloat16),
            pltpu.VMEM((N_DEV - 1, H, D), jnp.bfloat16),
            pltpu.VMEM((N_DEV - 1, H, D), jnp.bfloat16),
            pltpu.VMEM((N_DEV - 1, H, D), jnp.bfloat16),
            pltpu.VMEM((2, H, D), jnp.float32),
            pltpu.VMEM((2, H, D), jnp.float32),
            pltpu.SemaphoreType.DMA((N_DEV - 1, K)),
            pltpu.SemaphoreType.DMA((N_DEV - 1, K)),
            pltpu.SemaphoreType.DMA((N_DEV - 1, K)),
            pltpu.SemaphoreType.DMA((N_DEV - 1, K)),
            pltpu.SemaphoreType.DMA((2,)),
            pltpu.SemaphoreType.DMA((2,)),
        ],
        compiler_params=pltpu.CompilerParams(
            collective_id=0,
            vmem_limit_bytes=100 * 1024 * 1024,
        ),
    )(partial, gamma2)
